# Initial kernel scaffold; baseline (speedup 1.0000x reference)
import jax
import jax.numpy as jnp
from jax.experimental import pallas as pl


def kernel(text_input, embedding_weight):
    raise NotImplementedError("write your pallas kernel here")


# speedup vs baseline: 2.3369x; 2.3369x over previous
import functools

import jax
import jax.numpy as jnp
from jax import lax
from jax.experimental import pallas as pl
from jax.experimental.pallas import tpu as pltpu
from jax.experimental.pallas import tpu_sc as plsc

VOCAB_PLUS_OOV = 1001
DIM = 128
BATCH = 16384

NUM_CORES = 2
NUM_SUBCORES = 16
NUM_WORKERS = NUM_CORES * NUM_SUBCORES
ROWS_PER_WORKER = BATCH // NUM_WORKERS
CHUNK = 128
NUM_CHUNKS = ROWS_PER_WORKER // CHUNK

_mesh = plsc.VectorSubcoreMesh(core_axis_name="c", subcore_axis_name="s")


@functools.partial(
    pl.kernel,
    mesh=_mesh,
    out_type=jax.ShapeDtypeStruct((BATCH, DIM), jnp.float32),
    scratch_types=[
        pltpu.VMEM((NUM_CHUNKS, CHUNK), jnp.int32),
        pltpu.VMEM((ROWS_PER_WORKER, DIM), jnp.float32),
        pltpu.SemaphoreType.DMA,
    ],
)
def _embed_gather(idx_hbm, table_hbm, out_hbm, idx_v, rows_v, sem):
    wid = lax.axis_index("s") * NUM_CORES + lax.axis_index("c")
    base = wid * ROWS_PER_WORKER
    pltpu.sync_copy(idx_hbm.at[wid], idx_v)
    copies = [
        pltpu.async_copy(
            table_hbm.at[idx_v.at[j]],
            rows_v.at[pl.ds(j * CHUNK, CHUNK)],
            sem,
        )
        for j in range(NUM_CHUNKS)
    ]
    for c in copies:
        c.wait()
    pltpu.sync_copy(rows_v, out_hbm.at[pl.ds(base, ROWS_PER_WORKER)])


def kernel(text_input, embedding_weight):
    idx = text_input.astype(jnp.int32).reshape(NUM_WORKERS, NUM_CHUNKS, CHUNK)
    return _embed_gather(idx, embedding_weight)


# speedup vs baseline: 2.3523x; 1.0066x over previous
import functools

import jax
import jax.numpy as jnp
from jax import lax
from jax.experimental import pallas as pl
from jax.experimental.pallas import tpu as pltpu
from jax.experimental.pallas import tpu_sc as plsc

VOCAB_PLUS_OOV = 1001
DIM = 128
BATCH = 16384

NUM_CORES = 2
NUM_SUBCORES = 16
NUM_WORKERS = NUM_CORES * NUM_SUBCORES
ROWS_PER_WORKER = BATCH // NUM_WORKERS
CHUNK = 128
NUM_CHUNKS = ROWS_PER_WORKER // CHUNK

_mesh = plsc.VectorSubcoreMesh(core_axis_name="c", subcore_axis_name="s")


@functools.partial(
    pl.kernel,
    mesh=_mesh,
    out_type=jax.ShapeDtypeStruct((BATCH, DIM), jnp.float32),
    scratch_types=[
        pltpu.VMEM((NUM_CHUNKS, CHUNK), jnp.int32),
        pltpu.VMEM((ROWS_PER_WORKER, DIM), jnp.float32),
        pltpu.SemaphoreType.DMA((NUM_CHUNKS,)),
        pltpu.SemaphoreType.DMA,
    ],
)
def _embed_gather(idx_hbm, table_hbm, out_hbm, idx_v, rows_v, gsems, wsem):
    wid = lax.axis_index("s") * NUM_CORES + lax.axis_index("c")
    base = wid * ROWS_PER_WORKER
    pltpu.sync_copy(idx_hbm.at[wid], idx_v)
    gathers = [
        pltpu.async_copy(
            table_hbm.at[idx_v.at[j]],
            rows_v.at[pl.ds(j * CHUNK, CHUNK)],
            gsems.at[j],
        )
        for j in range(NUM_CHUNKS)
    ]
    writes = []
    for j in range(NUM_CHUNKS):
        gathers[j].wait()
        writes.append(
            pltpu.async_copy(
                rows_v.at[pl.ds(j * CHUNK, CHUNK)],
                out_hbm.at[pl.ds(base + j * CHUNK, CHUNK)],
                wsem,
            )
        )
    for w in writes:
        w.wait()


def kernel(text_input, embedding_weight):
    idx = text_input.astype(jnp.int32).reshape(NUM_WORKERS, NUM_CHUNKS, CHUNK)
    return _embed_gather(idx, embedding_weight)


# speedup vs baseline: 2.7504x; 1.1693x over previous
import functools

import jax
import jax.numpy as jnp
from jax import lax
from jax.experimental import pallas as pl
from jax.experimental.pallas import tpu as pltpu
from jax.experimental.pallas import tpu_sc as plsc

VOCAB_PLUS_OOV = 1001
DIM = 128
BATCH = 16384

NUM_CORES = 2
NUM_SUBCORES = 16
NUM_WORKERS = NUM_CORES * NUM_SUBCORES
ROWS_PER_WORKER = BATCH // NUM_WORKERS
CHUNK = 128
NUM_CHUNKS = ROWS_PER_WORKER // CHUNK

_mesh = plsc.VectorSubcoreMesh(core_axis_name="c", subcore_axis_name="s")


@functools.partial(
    pl.kernel,
    mesh=_mesh,
    out_type=jax.ShapeDtypeStruct((BATCH, DIM), jnp.float32),
    scratch_types=[
        pltpu.VMEM((NUM_CHUNKS, CHUNK), jnp.int32),
        pltpu.VMEM((ROWS_PER_WORKER, DIM), jnp.float32),
        pltpu.VMEM_SHARED((VOCAB_PLUS_OOV, DIM), jnp.float32),
        pltpu.SemaphoreType.DMA((NUM_CHUNKS,)),
        pltpu.SemaphoreType.DMA,
    ],
)
def _embed_gather(idx_hbm, table_hbm, out_hbm, idx_v, rows_v, table_sp, gsems, wsem):
    sid = lax.axis_index("s")
    wid = sid * NUM_CORES + lax.axis_index("c")
    base = wid * ROWS_PER_WORKER
    pltpu.sync_copy(idx_hbm.at[wid], idx_v)
    @pl.when(sid == 0)
    def _stage():
        pltpu.sync_copy(table_hbm, table_sp)

    plsc.subcore_barrier()
    gathers = [
        pltpu.async_copy(
            table_sp.at[idx_v.at[j]],
            rows_v.at[pl.ds(j * CHUNK, CHUNK)],
            gsems.at[j],
        )
        for j in range(NUM_CHUNKS)
    ]
    writes = []
    for j in range(NUM_CHUNKS):
        gathers[j].wait()
        writes.append(
            pltpu.async_copy(
                rows_v.at[pl.ds(j * CHUNK, CHUNK)],
                out_hbm.at[pl.ds(base + j * CHUNK, CHUNK)],
                wsem,
            )
        )
    for w in writes:
        w.wait()


def kernel(text_input, embedding_weight):
    idx = text_input.astype(jnp.int32).reshape(NUM_WORKERS, NUM_CHUNKS, CHUNK)
    return _embed_gather(idx, embedding_weight)
